# baseline (speedup 1.0000x reference)
import functools

import jax
import jax.numpy as jnp
from jax import lax
from jax.experimental import pallas as pl
from jax.experimental.pallas import tpu as pltpu
from jax.experimental.pallas import tpu_sc as plsc

def _affinity_body(w_ref, x_ref, out_t_ref):
    out_t_ref[...] = jax.lax.dot_general(
        w_ref[...], x_ref[...],
        (((1,), (1,)), ((), ())),
        preferred_element_type=jnp.float32,
    )


def _float_key(a):
    bits = jax.lax.bitcast_convert_type(a, jnp.int32)
    return jnp.where(bits >= 0, bits, bits ^ jnp.int32(0x7FFFFFFF))


def _stats_body(aff_ref, theta_ref, jcut_ref, mx_ref, denom_ref, *, C):
    a = aff_ref[...]
    E, T = a.shape
    key = _float_key(a)

    cnt_pos = jnp.sum((key >= 0).astype(jnp.int32), axis=1, keepdims=True)
    theta0 = jnp.where(cnt_pos >= C, 0, -2147483648).astype(jnp.int32)

    def _count_ge(cand):
        return jnp.sum((key >= cand).astype(jnp.int32), axis=1, keepdims=True)

    def _theta_step2(i, prefix):
        hi = jnp.int32(1) << (30 - 2 * i)
        lo = jnp.int32(1) << (29 - 2 * i)
        c1 = prefix + lo
        c2 = prefix + hi
        c3 = prefix + hi + lo
        n1, n2, n3 = _count_ge(c1), _count_ge(c2), _count_ge(c3)
        return jnp.where(n3 >= C, c3,
                         jnp.where(n2 >= C, c2,
                                   jnp.where(n1 >= C, c1, prefix)))

    theta = jax.lax.fori_loop(0, 15, _theta_step2, theta0)
    cand = theta + 1
    theta = jnp.where(_count_ge(cand) >= C, cand, theta)

    gt = key > theta
    eq = key == theta
    n_gt = jnp.sum(gt.astype(jnp.int32), axis=1, keepdims=True)
    n_eq = jnp.sum(eq.astype(jnp.int32), axis=1, keepdims=True)
    m = C - n_gt

    def _j_fast(_):
        return jnp.full((E, 1), T, jnp.int32)

    idx = jax.lax.broadcasted_iota(jnp.int32, (E, T), 1)

    def _j_search(_):
        def _j_step(i, J):
            cand_j = J + (jnp.int32(1) << (15 - i))
            cnt = jnp.sum((eq & (idx < cand_j)).astype(jnp.int32),
                          axis=1, keepdims=True)
            return jnp.where(cnt <= m, cand_j, J)

        return jax.lax.fori_loop(0, 16, _j_step,
                                 jnp.zeros((E, 1), jnp.int32))

    jcut = jax.lax.cond(jnp.all(n_eq == m), _j_fast, _j_search, 0)

    sel = gt | (eq & (idx < jcut))
    mx = jnp.max(a, axis=1, keepdims=True)
    denom = jnp.sum(jnp.where(sel, jnp.exp(a - mx), 0.0),
                    axis=1, keepdims=True)

    theta_ref[...] = theta
    jcut_ref[...] = jcut
    mx_ref[...] = mx
    denom_ref[...] = denom


def _sc_stats_body(aff_hbm, out_hbm, va, hist, ovec, *, C, T):
    wid = lax.axis_index("s") * 2 + lax.axis_index("c")
    nch8 = T // 128

    def _keys(base, u):
        return lax.bitcast_convert_type(va[pl.ds(base + u * 16, 16)], jnp.int32)

    results = []
    for j in range(2):
        e = 2 * wid + j
        pltpu.sync_copy(aff_hbm.at[e], va)

        def _pre(i, carry):
            mxv, posv = carry
            base = i * 128
            for u in range(8):
                c = va[pl.ds(base + u * 16, 16)]
                b = lax.bitcast_convert_type(c, jnp.int32)
                k = jnp.where(b >= 0, b, b ^ 0x7FFFFFFF)
                va[pl.ds(base + u * 16, 16)] = lax.bitcast_convert_type(k, jnp.float32)
                mxv = jnp.maximum(mxv, c)
                posv = posv + jnp.where(k >= 0, 1, 0)
            return mxv, posv

        mxv, posv = lax.fori_loop(
            0, nch8, _pre,
            (jnp.full((16,), -3.0e38, jnp.float32),
             jnp.zeros((16,), jnp.int32)))
        mxf = jnp.max(mxv)

        def _hist_level(shift, prefix, match_shift):
            def _z(i, _):
                hist[pl.ds(i * 16, 16)] = jnp.zeros((16,), jnp.int32)
                return 0

            lax.fori_loop(0, 64, _z, 0)
            ones = jnp.ones((16,), jnp.int32)

            def _acc(i, _):
                base = i * 128
                for u in range(8):
                    k = _keys(base, u)
                    if match_shift is None:
                        d = (k >> shift) + 512
                        plsc.addupdate_scatter(hist, [d], ones)
                    else:
                        valid = ((k >> match_shift)
                                 == (prefix >> match_shift))
                        d = (k >> shift) & 1023
                        plsc.addupdate_scatter(hist, [d], ones, mask=valid)
                return 0

            lax.fori_loop(0, nch8, _acc, 0)

        def _find_bucket(need):
            def _cs(i, carry):
                cum, dchunk, cum_above = carry
                jj = 63 - i
                s = jnp.sum(hist[pl.ds(jj * 16, 16)])
                newcum = cum + s
                crossed = (cum < need) & (newcum >= need)
                dchunk = jnp.where(crossed, jj, dchunk)
                cum_above = jnp.where(crossed, cum, cum_above)
                return newcum, dchunk, cum_above

            _, dchunk, cum_above = lax.fori_loop(
                0, 64, _cs,
                (jnp.int32(0), jnp.int32(0), jnp.int32(0)))
            h = hist[pl.ds(dchunk * 16, 16)]
            csum = plsc.cumsum(h)
            total = jnp.max(csum)
            suff = cum_above + total - csum + h
            npop = jnp.max(plsc.all_reduce_population_count(suff >= need))
            lstar = npop - 1
            lanes16 = lax.iota(jnp.int32, 16)
            csum_l = jnp.sum(jnp.where(lanes16 == lstar, csum, 0))
            d = dchunk * 16 + lstar
            return d, cum_above + total - csum_l

        _hist_level(22, jnp.int32(0), None)
        d1, n_hi = _find_bucket(jnp.int32(C))
        prefix = (d1 - 512) << 22
        _hist_level(12, prefix, 22)
        d2, r2 = _find_bucket(C - n_hi)
        n_hi = n_hi + r2
        prefix = prefix | (d2 << 12)
        _hist_level(2, prefix, 12)
        d3, r3 = _find_bucket(C - n_hi)
        prefix = prefix | (d3 << 2)

        def _count_ge(cand):
            def _cp(i, acc):
                base = i * 128
                for u in range(8):
                    k = _keys(base, u)
                    acc = acc + jnp.where(k >= cand, 1, 0)
                return acc

            return jnp.sum(lax.fori_loop(0, nch8, _cp,
                                         jnp.zeros((16,), jnp.int32)))

        for b in (1, 0):
            cand = prefix + (1 << b)
            prefix = jnp.where(_count_ge(cand) >= C, cand, prefix)
        theta = prefix

        def _fin(i, carry):
            g, q, se = carry
            base = i * 128
            for u in range(8):
                k = _keys(base, u)
                b2 = jnp.where(k >= 0, k, k ^ 0x7FFFFFFF)
                a = lax.bitcast_convert_type(b2, jnp.float32)
                gtm = k > theta
                g = g + jnp.where(gtm, 1, 0)
                q = q + jnp.where(k == theta, 1, 0)
                se = se + jnp.where(gtm, jnp.exp(a - mxf), 0.0)
            return g, q, se

        g, q, se = lax.fori_loop(
            0, nch8, _fin,
            (jnp.zeros((16,), jnp.int32), jnp.zeros((16,), jnp.int32),
             jnp.zeros((16,), jnp.float32)))
        n_gt = jnp.sum(g)
        n_eq = jnp.sum(q)
        m = C - n_gt
        tb = jnp.zeros((16,), jnp.int32) + theta
        tf = lax.bitcast_convert_type(jnp.where(tb >= 0, tb, tb ^ 0x7FFFFFFF),
                          jnp.float32)
        exp_theta = jnp.max(jnp.exp(tf - mxf))
        denom = (jnp.sum(se)
                 + lax.convert_element_type(m, jnp.float32) * exp_theta)

        def _fastj(_):
            return jnp.int32(T)

        def _searchj(_):
            def _jstep(s, J):
                candj = J + (jnp.int32(1) << (15 - s))

                def _cp(i, acc):
                    base = i * 128
                    for u in range(8):
                        k = _keys(base, u)
                        idxv = base + u * 16 + lax.iota(jnp.int32, 16)
                        acc = acc + jnp.where(
                            (k == theta) & (idxv < candj), 1, 0)
                    return acc

                cnt = jnp.sum(lax.fori_loop(0, nch8, _cp,
                                            jnp.zeros((16,), jnp.int32)))
                return jnp.where(cnt <= m, candj, J)

            return lax.fori_loop(0, 16, _jstep, jnp.int32(0))

        jcut = lax.cond(n_eq == m, _fastj, _searchj, 0)
        results.append((theta, jcut, mxf, denom))

    lanes = lax.iota(jnp.int32, 16)
    vi = jnp.where(lanes == 0, results[0][0],
                   jnp.where(lanes == 1, results[1][0],
                             jnp.where(lanes == 2, results[0][1],
                                       results[1][1])))
    vf = jnp.where(lanes == 4, results[0][2],
                   jnp.where(lanes == 5, results[1][2],
                             jnp.where(lanes == 6, results[0][3],
                                       results[1][3])))
    ovec[...] = jnp.where(lanes < 4, lax.bitcast_convert_type(vi, jnp.float32), vf)
    pltpu.sync_copy(ovec, out_hbm.at[wid])


def _sc_stats_call(aff_t, C):
    n_exp, T = aff_t.shape
    mesh = plsc.VectorSubcoreMesh(core_axis_name="c", subcore_axis_name="s")
    run = pl.kernel(
        functools.partial(_sc_stats_body, C=C, T=T),
        mesh=mesh,
        compiler_params=pltpu.CompilerParams(needs_layout_passes=False),
        out_type=jax.ShapeDtypeStruct((n_exp // 2, 16), jnp.float32),
        scratch_types=[
            pltpu.VMEM((T,), jnp.float32),
            pltpu.VMEM((1024,), jnp.int32),
            pltpu.VMEM((16,), jnp.float32),
        ],
    )
    packed = run(aff_t)
    pi = lax.bitcast_convert_type(packed[:, 0:4], jnp.int32)
    theta = pi[:, 0:2].reshape(n_exp, 1)
    jcut = pi[:, 2:4].reshape(n_exp, 1)
    mx = packed[:, 4:6].reshape(n_exp, 1)
    denom = packed[:, 6:8].reshape(n_exp, 1)
    return theta, jcut, mx, denom


def _emit_body(aff_ref, theta_ref, jcut_ref, mx_ref, denom_ref,
               w_out_ref, a_out_ref, *, bt):
    i = pl.program_id(0)
    a = aff_ref[...]
    E = a.shape[0]
    key = _float_key(a)
    theta = theta_ref[...]
    idx = i * bt + jax.lax.broadcasted_iota(jnp.int32, (E, bt), 1)
    sel = (key > theta) | ((key == theta) & (idx < jcut_ref[...]))
    ex = jnp.exp(a - mx_ref[...])
    w_un = jnp.where(sel, ex / denom_ref[...], 0.0)
    cnt = jnp.sum(sel.astype(jnp.float32), axis=0, keepdims=True)
    w = w_un / jnp.maximum(cnt, 1.0)
    w_out_ref[...] = w.T
    a_out_ref[...] = sel.astype(jnp.float32).T


def kernel(hidden_states, W_sel):
    batch, seq, d_model = hidden_states.shape
    n_exp = W_sel.shape[0]
    num_tokens = batch * seq
    capacity = int(num_tokens * 1.0 / n_exp)
    C = min(capacity, num_tokens)

    x = hidden_states.reshape(num_tokens, d_model)

    bt = 4096
    aff_t = pl.pallas_call(
        _affinity_body,
        grid=(num_tokens // bt,),
        in_specs=[
            pl.BlockSpec((n_exp, d_model), lambda i: (0, 0)),
            pl.BlockSpec((bt, d_model), lambda i: (i, 0)),
        ],
        out_specs=pl.BlockSpec((n_exp, bt), lambda i: (0, i)),
        out_shape=jax.ShapeDtypeStruct((n_exp, num_tokens), jnp.float32),
    )(W_sel, x)

    theta, jcut, mx, denom = _sc_stats_call(aff_t, C)

    bt2 = 4096
    full_stat = pl.BlockSpec((n_exp, 1), lambda i: (0, 0))
    weights, assignments = pl.pallas_call(
        functools.partial(_emit_body, bt=bt2),
        grid=(num_tokens // bt2,),
        in_specs=[
            pl.BlockSpec((n_exp, bt2), lambda i: (0, i)),
            full_stat, full_stat, full_stat, full_stat,
        ],
        out_specs=(
            pl.BlockSpec((bt2, n_exp), lambda i: (i, 0)),
            pl.BlockSpec((bt2, n_exp), lambda i: (i, 0)),
        ),
        out_shape=(
            jax.ShapeDtypeStruct((num_tokens, n_exp), jnp.float32),
            jax.ShapeDtypeStruct((num_tokens, n_exp), jnp.float32),
        ),
    )(aff_t, theta, jcut, mx, denom)

    return weights, assignments, capacity


# speedup vs baseline: 1.6884x; 1.6884x over previous
import functools

import jax
import jax.numpy as jnp
from jax.experimental import pallas as pl

def _affinity_body(w_ref, x_ref, out_t_ref):
    out_t_ref[...] = jax.lax.dot_general(
        w_ref[...], x_ref[...],
        (((1,), (1,)), ((), ())),
        preferred_element_type=jnp.float32,
    )


def _float_key(a):
    bits = jax.lax.bitcast_convert_type(a, jnp.int32)
    return jnp.where(bits >= 0, bits, bits ^ jnp.int32(0x7FFFFFFF))


def _stats_body(aff_ref, theta_ref, jcut_ref, mx_ref, denom_ref, *, C):
    a = aff_ref[...]
    E, T = a.shape
    key = _float_key(a)

    cnt_pos = jnp.sum((key >= 0).astype(jnp.int32), axis=1, keepdims=True)
    theta0 = jnp.where(cnt_pos >= C, 0, -2147483648).astype(jnp.int32)

    def _count_ge(cand):
        return jnp.sum((key >= cand).astype(jnp.int32), axis=1, keepdims=True)

    def _theta_step2(i, prefix):
        hi = jnp.int32(1) << (30 - 2 * i)
        lo = jnp.int32(1) << (29 - 2 * i)
        c1 = prefix + lo
        c2 = prefix + hi
        c3 = prefix + hi + lo
        n1, n2, n3 = _count_ge(c1), _count_ge(c2), _count_ge(c3)
        return jnp.where(n3 >= C, c3,
                         jnp.where(n2 >= C, c2,
                                   jnp.where(n1 >= C, c1, prefix)))

    theta = jax.lax.fori_loop(0, 15, _theta_step2, theta0)
    cand = theta + 1
    theta = jnp.where(_count_ge(cand) >= C, cand, theta)

    gt = key > theta
    eq = key == theta
    n_gt = jnp.sum(gt.astype(jnp.int32), axis=1, keepdims=True)
    n_eq = jnp.sum(eq.astype(jnp.int32), axis=1, keepdims=True)
    m = C - n_gt

    def _j_fast(_):
        return jnp.full((E, 1), T, jnp.int32)

    idx = jax.lax.broadcasted_iota(jnp.int32, (E, T), 1)

    def _j_search(_):
        def _j_step(i, J):
            cand_j = J + (jnp.int32(1) << (15 - i))
            cnt = jnp.sum((eq & (idx < cand_j)).astype(jnp.int32),
                          axis=1, keepdims=True)
            return jnp.where(cnt <= m, cand_j, J)

        return jax.lax.fori_loop(0, 16, _j_step,
                                 jnp.zeros((E, 1), jnp.int32))

    jcut = jax.lax.cond(jnp.all(n_eq == m), _j_fast, _j_search, 0)

    sel = gt | (eq & (idx < jcut))
    mx = jnp.max(a, axis=1, keepdims=True)
    denom = jnp.sum(jnp.where(sel, jnp.exp(a - mx), 0.0),
                    axis=1, keepdims=True)

    theta_ref[...] = theta
    jcut_ref[...] = jcut
    mx_ref[...] = mx
    denom_ref[...] = denom


def _emit_body(aff_ref, theta_ref, jcut_ref, mx_ref, denom_ref,
               w_out_ref, a_out_ref, *, bt):
    i = pl.program_id(0)
    a = aff_ref[...]
    E = a.shape[0]
    key = _float_key(a)
    theta = theta_ref[...]
    idx = i * bt + jax.lax.broadcasted_iota(jnp.int32, (E, bt), 1)
    sel = (key > theta) | ((key == theta) & (idx < jcut_ref[...]))
    ex = jnp.exp(a - mx_ref[...])
    w_un = jnp.where(sel, ex / denom_ref[...], 0.0)
    cnt = jnp.sum(sel.astype(jnp.float32), axis=0, keepdims=True)
    w = w_un / jnp.maximum(cnt, 1.0)
    w_out_ref[...] = w.T
    a_out_ref[...] = sel.astype(jnp.float32).T


def kernel(hidden_states, W_sel):
    batch, seq, d_model = hidden_states.shape
    n_exp = W_sel.shape[0]
    num_tokens = batch * seq
    capacity = int(num_tokens * 1.0 / n_exp)
    C = min(capacity, num_tokens)

    x = hidden_states.reshape(num_tokens, d_model)

    bt = 8192
    aff_t = pl.pallas_call(
        _affinity_body,
        grid=(num_tokens // bt,),
        in_specs=[
            pl.BlockSpec((n_exp, d_model), lambda i: (0, 0)),
            pl.BlockSpec((bt, d_model), lambda i: (i, 0)),
        ],
        out_specs=pl.BlockSpec((n_exp, bt), lambda i: (0, i)),
        out_shape=jax.ShapeDtypeStruct((n_exp, num_tokens), jnp.float32),
    )(W_sel, x)

    stat_i32 = jax.ShapeDtypeStruct((n_exp, 1), jnp.int32)
    stat_f32 = jax.ShapeDtypeStruct((n_exp, 1), jnp.float32)
    theta, jcut, mx, denom = pl.pallas_call(
        functools.partial(_stats_body, C=C),
        out_shape=(stat_i32, stat_i32, stat_f32, stat_f32),
    )(aff_t)

    bt2 = 8192
    full_stat = pl.BlockSpec((n_exp, 1), lambda i: (0, 0))
    weights, assignments = pl.pallas_call(
        functools.partial(_emit_body, bt=bt2),
        grid=(num_tokens // bt2,),
        in_specs=[
            pl.BlockSpec((n_exp, bt2), lambda i: (0, i)),
            full_stat, full_stat, full_stat, full_stat,
        ],
        out_specs=(
            pl.BlockSpec((bt2, n_exp), lambda i: (i, 0)),
            pl.BlockSpec((bt2, n_exp), lambda i: (i, 0)),
        ),
        out_shape=(
            jax.ShapeDtypeStruct((num_tokens, n_exp), jnp.float32),
            jax.ShapeDtypeStruct((num_tokens, n_exp), jnp.float32),
        ),
    )(aff_t, theta, jcut, mx, denom)

    return weights, assignments, capacity
